# baseline (speedup 1.0000x reference)
import functools

import jax
import jax.numpy as jnp
from jax import lax
from jax.experimental import pallas as pl
from jax.experimental.pallas import tpu as pltpu
from jax.experimental.pallas import tpu_sc as plsc

_B = 8
_C = 21
_H = 512
_W = 512
_R = 128

_NW = 32
_PROWS = _B * _H
_ROWS_PER_W = _PROWS // _NW
_SUBROWS = 16


def _maxarg_body(x_ref, p_ref):
    x = x_ref[0]
    m = x[0]
    a = jnp.zeros(m.shape, jnp.int32)
    for c in range(1, _C):
        xc = x[c]
        upd = xc > m
        m = jnp.where(upd, xc, m)
        a = jnp.where(upd, c, a)
    mi = lax.bitcast_convert_type(m, jnp.int32)
    p_ref[0] = (mi & -32) | a


def _stage1(x):
    return pl.pallas_call(
        _maxarg_body,
        grid=(_B, _H // _R),
        in_specs=[pl.BlockSpec((1, _C, _R, _W), lambda b, t: (b, 0, t, 0))],
        out_specs=pl.BlockSpec((1, _R, _W), lambda b, t: (b, t, 0)),
        out_shape=jax.ShapeDtypeStruct((_B, _H, _W), jnp.int32),
    )(x)


def _make_stage2():
    mesh = plsc.VectorSubcoreMesh(core_axis_name="c", subcore_axis_name="s")

    @functools.partial(
        pl.kernel,
        mesh=mesh,
        out_type=jax.ShapeDtypeStruct((_NW, 32), jnp.float32),
        compiler_params=pltpu.CompilerParams(needs_layout_passes=False),
        scratch_types=[
            pltpu.VMEM((_SUBROWS, _W), jnp.int32),
            pltpu.VMEM((_SUBROWS, _W), jnp.int32),
            pltpu.VMEM((16, 32), jnp.float32),
            pltpu.VMEM((32,), jnp.float32),
            pltpu.SemaphoreType.DMA,
            pltpu.SemaphoreType.DMA,
        ],
    )
    def segsum(p_hbm, out_hbm, pv0, pv1, acc2, acc, sem0, sem1):
        wid = lax.axis_index("s") * 2 + lax.axis_index("c")
        base = wid * _ROWS_PER_W
        bufs = (pv0, pv1)
        sems = (sem0, sem1)
        zeros = jnp.zeros((16,), jnp.float32)
        for r in range(16):
            acc2[r, pl.ds(0, 16)] = zeros
            acc2[r, pl.ds(16, 16)] = zeros
        rows = lax.iota(jnp.int32, 16)
        n_sub = _ROWS_PER_W // _SUBROWS

        copies = [
            pltpu.make_async_copy(
                p_hbm.at[pl.ds(base + s * _SUBROWS, _SUBROWS), :],
                bufs[s % 2],
                sems[s % 2],
            )
            for s in range(n_sub)
        ]
        copies[0].start()
        for sub in range(n_sub):
            if sub + 1 < n_sub:
                copies[sub + 1].start()
            copies[sub].wait()
            buf = bufs[sub % 2]

            def body(r, carry, buf=buf):
                for u in range(_W // 16):
                    p = buf[r, pl.ds(u * 16, 16)]
                    ids = p & 31
                    vals = plsc.bitcast(p & -32, jnp.float32)
                    plsc.addupdate_scatter(acc2, [rows, ids], vals)
                return carry

            lax.fori_loop(0, _SUBROWS, body, 0)

        lo = acc2[0, pl.ds(0, 16)]
        hi = acc2[0, pl.ds(16, 16)]
        for r in range(1, 16):
            lo = lo + acc2[r, pl.ds(0, 16)]
            hi = hi + acc2[r, pl.ds(16, 16)]
        acc[pl.ds(0, 16)] = lo
        acc[pl.ds(16, 16)] = hi
        pltpu.sync_copy(acc, out_hbm.at[wid])

    return segsum


@functools.cache
def _stage2():
    return _make_stage2()


def kernel(x):
    p = _stage1(x)
    partials = _stage2()(p.reshape(_PROWS, _W))
    return partials.reshape(_B, _NW // _B, 32).sum(axis=1)[:, :_C]
